# baseline (speedup 1.0000x reference)
import functools

import jax
import jax.numpy as jnp
from jax import lax
from jax.experimental import pallas as pl
from jax.experimental.pallas import tpu as pltpu


def _sigmoid(x):
    return 0.5 * jnp.tanh(0.5 * x) + 0.5


def _dot_tb(a, w):
    return lax.dot_general(a, w, (((1,), (1,)), ((), ())),
                           preferred_element_type=jnp.float32)


def _make_body(L, H, TB, F):
    def body(x_ref, wi1, wh1, bi1, bhn1, wi2, wh2, bi2, bhn2, wout, bout,
             out_ref, act_ref):
        bi1v = bi1[...]
        bhn1v = bhn1[...]
        bhn2v = bhn2[...]
        bi2v = bi2[...]
        wi1v = wi1[...].astype(jnp.bfloat16)
        wh1v = wh1[...].astype(jnp.bfloat16)
        wh2v = wh2[...].astype(jnp.bfloat16)
        wi2v = wi2[...].astype(jnp.bfloat16)

        def cell(gi, h, wh, bhn):
            gh = _dot_tb(h.astype(jnp.bfloat16), wh)
            rz = _sigmoid(gi[:, :2 * H] + gh[:, :2 * H])
            r = rz[:, :H]
            z = rz[:, H:]
            n = jnp.tanh(gi[:, 2 * H:] + r * (gh[:, 2 * H:] + bhn))
            return n + z * (h - n)

        def step1(t, h1):
            gi = _dot_tb(x_ref[t].astype(jnp.bfloat16), wi1v) + bi1v
            return cell(gi, h1, wh1v, bhn1v)

        def step2(h1_in, h2):
            gi = _dot_tb(h1_in.astype(jnp.bfloat16), wi2v) + bi2v
            return cell(gi, h2, wh2v, bhn2v)

        zero_h = jnp.zeros((TB, H), jnp.float32)
        h1 = step1(0, zero_h)
        h2 = zero_h
        for t in range(1, L):
            h1_next = step1(t, h1)
            h2 = step2(h1, h2)
            act_ref[t - 1] = h2.astype(jnp.bfloat16)
            h1 = h1_next
        h2 = step2(h1, h2)
        act_ref[L - 1] = h2.astype(jnp.bfloat16)

        w = wout[...]
        logits = jnp.sum(act_ref[...].astype(jnp.float32) * w[None], axis=2)
        out_ref[...] = _sigmoid(logits + bout[...])

    return body


def _round_up(x, m):
    return (x + m - 1) // m * m


@jax.jit
def kernel(x, kp0, kp1, kp2, kp3, kp4, kp5, kp6, kp7, kp8, kp9):
    L, B, F = x.shape
    H = kp8.shape[0]
    TB = 256 if B % 256 == 0 else _round_up(min(B, 256), 8)
    B_pad = _round_up(B, TB)

    xb = x
    if B_pad != B:
        xb = jnp.pad(xb, ((0, 0), (0, B_pad - B), (0, 0)))

    bi1 = kp2.reshape(1, 3 * H)
    bhn1 = kp3.reshape(1, H)
    bi2 = kp6.reshape(1, 3 * H)
    bhn2 = kp7.reshape(1, H)
    wout = kp8.reshape(1, H)
    bout = kp9

    params = [kp0, kp1, bi1, bhn1, kp4, kp5, bi2, bhn2, wout, bout]
    w_specs = [pl.BlockSpec(p.shape, lambda i, nd=p.ndim: (0,) * nd)
               for p in params]

    grid = (B_pad // TB,)
    out = pl.pallas_call(
        _make_body(L, H, TB, F),
        out_shape=jax.ShapeDtypeStruct((L, B_pad), jnp.float32),
        grid=grid,
        in_specs=[pl.BlockSpec((L, TB, F), lambda i: (0, i, 0))] + w_specs,
        out_specs=pl.BlockSpec((L, TB), lambda i: (0, i)),
        scratch_shapes=[pltpu.VMEM((L, TB, H), jnp.bfloat16)],
        compiler_params=pltpu.CompilerParams(
            dimension_semantics=("parallel",),
            vmem_limit_bytes=64 * 1024 * 1024),
    )(xb, *params)

    if B_pad != B:
        out = out[:, :B]
    return out[:, :, None]


# speedup vs baseline: 1.0568x; 1.0568x over previous
import functools

import jax
import jax.numpy as jnp
from jax import lax
from jax.experimental import pallas as pl
from jax.experimental.pallas import tpu as pltpu


def _sigmoid(x):
    return 0.5 * jnp.tanh(0.5 * x) + 0.5


def _dot_tb(a, w):
    return lax.dot_general(a, w, (((1,), (1,)), ((), ())),
                           preferred_element_type=jnp.float32)


def _make_body(L, H, TB, F):
    def body(x_ref, wi1, wh1, bi1, bhn1, wi2, wh2, bi2, bhn2, wout, bout,
             out_ref, act_ref):
        bi1v = bi1[...]
        bhn1v = bhn1[...]
        bhn2v = bhn2[...]
        bi2v = bi2[...]
        wi1v = wi1[...]
        wh1v = wh1[...]
        wh2v = wh2[...]
        wi2v = wi2[...]

        def cell(gi, h, wh, bhn):
            gh = _dot_tb(h, wh)
            rz = _sigmoid(gi[:, :2 * H] + gh[:, :2 * H])
            r = rz[:, :H]
            z = rz[:, H:]
            n = jnp.tanh(gi[:, 2 * H:] + r * (gh[:, 2 * H:] + bhn))
            return n + z * (h - n)

        def step1(t, h1):
            gi = _dot_tb(x_ref[t], wi1v) + bi1v
            return cell(gi, h1, wh1v, bhn1v)

        def step2(h1_in, h2):
            gi = _dot_tb(h1_in, wi2v) + bi2v
            return cell(gi, h2, wh2v, bhn2v)

        zero_h = jnp.zeros((TB, H), jnp.float32)
        h1 = step1(0, zero_h)
        h2 = zero_h
        for t in range(1, L):
            h1_next = step1(t, h1)
            h2 = step2(h1, h2)
            act_ref[t - 1] = h2.astype(jnp.bfloat16)
            h1 = h1_next
        h2 = step2(h1, h2)
        act_ref[L - 1] = h2.astype(jnp.bfloat16)

        w = wout[...]
        logits = jnp.sum(act_ref[...].astype(jnp.float32) * w[None], axis=2)
        out_ref[...] = _sigmoid(logits + bout[...])

    return body


def _round_up(x, m):
    return (x + m - 1) // m * m


@jax.jit
def kernel(x, kp0, kp1, kp2, kp3, kp4, kp5, kp6, kp7, kp8, kp9):
    L, B, F = x.shape
    H = kp8.shape[0]
    TB = 256 if B % 256 == 0 else _round_up(min(B, 256), 8)
    B_pad = _round_up(B, TB)

    xb = x
    if B_pad != B:
        xb = jnp.pad(xb, ((0, 0), (0, B_pad - B), (0, 0)))

    bi1 = kp2.reshape(1, 3 * H)
    bhn1 = kp3.reshape(1, H)
    bi2 = kp6.reshape(1, 3 * H)
    bhn2 = kp7.reshape(1, H)
    wout = kp8.reshape(1, H)
    bout = kp9

    params = [kp0, kp1, bi1, bhn1, kp4, kp5, bi2, bhn2, wout, bout]
    w_specs = [pl.BlockSpec(p.shape, lambda i, nd=p.ndim: (0,) * nd)
               for p in params]

    grid = (B_pad // TB,)
    out = pl.pallas_call(
        _make_body(L, H, TB, F),
        out_shape=jax.ShapeDtypeStruct((L, B_pad), jnp.float32),
        grid=grid,
        in_specs=[pl.BlockSpec((L, TB, F), lambda i: (0, i, 0))] + w_specs,
        out_specs=pl.BlockSpec((L, TB), lambda i: (0, i)),
        scratch_shapes=[pltpu.VMEM((L, TB, H), jnp.bfloat16)],
        compiler_params=pltpu.CompilerParams(
            dimension_semantics=("parallel",),
            vmem_limit_bytes=64 * 1024 * 1024),
    )(xb, *params)

    if B_pad != B:
        out = out[:, :B]
    return out[:, :, None]
